# baseline (speedup 1.0000x reference)
import jax
import jax.numpy as jnp
from jax import lax
from jax.experimental import pallas as pl
from jax.experimental.pallas import tpu as pltpu
from jax.experimental.pallas import tpu_sc as plsc

B = 16384
VOCAB = 1000
EMB = 128

NUM_CORES = 2
NUM_SUBCORES = 16
LANES = 16
NUM_WORKERS = NUM_CORES * NUM_SUBCORES
B_PER_W = B // NUM_WORKERS
CHUNK = 128
N_CHUNKS = B_PER_W // CHUNK
HALF = B_PER_W // 2


def _sc_body(table_hbm, tok_hbm, out_hbm, idx_v, rows_v, sem_a, sem_b, sem_o):
    wid = lax.axis_index("s") * NUM_CORES + lax.axis_index("c")
    base = wid * B_PER_W

    pltpu.sync_copy(tok_hbm.at[pl.ds(base, B_PER_W)], idx_v)

    def remap(i, carry):
        t = idx_v[pl.ds(i * LANES, LANES)]
        ok = (t >= 0) & (t < VOCAB)
        idx_v[pl.ds(i * LANES, LANES)] = jnp.where(ok, t + 1, 0)
        return carry

    def gathers(lo_chunk, n, sem):
        return [
            pltpu.async_copy(
                table_hbm.at[idx_v.at[pl.ds((lo_chunk + j) * CHUNK, CHUNK)]],
                rows_v.at[pl.ds((lo_chunk + j) * CHUNK, CHUNK)],
                sem,
            )
            for j in range(n)
        ]

    lax.fori_loop(0, HALF // LANES, remap, 0, unroll=4)
    g_a = gathers(0, N_CHUNKS // 2, sem_a)
    lax.fori_loop(HALF // LANES, B_PER_W // LANES, remap, 0, unroll=4)
    g_b = gathers(N_CHUNKS // 2, N_CHUNKS // 2, sem_b)

    for c in g_a:
        c.wait()
    o_a = pltpu.async_copy(
        rows_v.at[pl.ds(0, HALF)], out_hbm.at[pl.ds(base, HALF)], sem_o)
    for c in g_b:
        c.wait()
    o_b = pltpu.async_copy(
        rows_v.at[pl.ds(HALF, HALF)], out_hbm.at[pl.ds(base + HALF, HALF)],
        sem_o)
    o_a.wait()
    o_b.wait()


@jax.jit
def kernel(table, inputs):
    tokens = inputs.reshape(B).astype(jnp.int32)
    mesh = plsc.VectorSubcoreMesh(core_axis_name="c", subcore_axis_name="s")
    run = pl.kernel(
        _sc_body,
        out_type=jax.ShapeDtypeStruct((B, EMB), jnp.float32),
        mesh=mesh,
        scratch_types=[
            pltpu.VMEM((B_PER_W,), jnp.int32),
            pltpu.VMEM((B_PER_W, EMB), jnp.float32),
            pltpu.SemaphoreType.DMA,
            pltpu.SemaphoreType.DMA,
            pltpu.SemaphoreType.DMA,
        ],
    )
    return run(table, tokens)


# speedup vs baseline: 1.0198x; 1.0198x over previous
import jax
import jax.numpy as jnp
from jax import lax
from jax.experimental import pallas as pl
from jax.experimental.pallas import tpu as pltpu
from jax.experimental.pallas import tpu_sc as plsc

B = 16384
VOCAB = 1000
EMB = 128

NUM_CORES = 2
NUM_SUBCORES = 16
LANES = 16
NUM_WORKERS = NUM_CORES * NUM_SUBCORES
B_PER_W = B // NUM_WORKERS
CHUNK = 128
N_CHUNKS = B_PER_W // CHUNK


def _sc_body(table_hbm, tok_hbm, out_hbm, idx_v, rows_v, sem):
    wid = lax.axis_index("s") * NUM_CORES + lax.axis_index("c")
    base = wid * B_PER_W

    pltpu.sync_copy(tok_hbm.at[pl.ds(base, B_PER_W)], idx_v)

    shifted = table_hbm.at[pl.ds(8, VOCAB)]
    copies = []
    for j in range(N_CHUNKS):
        copies.append(
            pltpu.async_copy(
                shifted.at[idx_v.at[pl.ds(j * CHUNK, CHUNK)]],
                rows_v.at[pl.ds(j * CHUNK, CHUNK)],
                sem,
            )
        )
    for c in copies:
        c.wait()

    pltpu.sync_copy(rows_v, out_hbm.at[pl.ds(base, B_PER_W)])


@jax.jit
def kernel(table, inputs):
    tokens = inputs.reshape(B).astype(jnp.int32)
    padded = jnp.concatenate([jnp.zeros((7, EMB), jnp.float32), table])
    mesh = plsc.VectorSubcoreMesh(core_axis_name="c", subcore_axis_name="s")
    run = pl.kernel(
        _sc_body,
        out_type=jax.ShapeDtypeStruct((B, EMB), jnp.float32),
        mesh=mesh,
        scratch_types=[
            pltpu.VMEM((B_PER_W,), jnp.int32),
            pltpu.VMEM((B_PER_W, EMB), jnp.float32),
            pltpu.SemaphoreType.DMA,
        ],
    )
    return run(padded, tokens)


# speedup vs baseline: 1.0257x; 1.0058x over previous
import jax
import jax.numpy as jnp
from jax import lax
from jax.experimental import pallas as pl
from jax.experimental.pallas import tpu as pltpu
from jax.experimental.pallas import tpu_sc as plsc

B = 16384
VOCAB = 1000
EMB = 128

NUM_CORES = 2
NUM_SUBCORES = 16
LANES = 16
NUM_WORKERS = NUM_CORES * NUM_SUBCORES
B_PER_W = B // NUM_WORKERS
CHUNK = 128
N_CHUNKS = B_PER_W // CHUNK


def _sc_body(table_hbm, tok_hbm, out_hbm, idx_v, rows_v, sem):
    wid = lax.axis_index("s") * NUM_CORES + lax.axis_index("c")
    base = wid * B_PER_W

    pltpu.sync_copy(tok_hbm.at[pl.ds(base, B_PER_W)], idx_v)

    def remap(i, carry):
        t = idx_v[pl.ds(i * LANES, LANES)]
        ok = (t >= 0) & (t < VOCAB)
        idx_v[pl.ds(i * LANES, LANES)] = jnp.where(ok, t + 1, 0)
        return carry

    lax.fori_loop(0, B_PER_W // LANES, remap, 0, unroll=4)

    copies = []
    for j in range(N_CHUNKS):
        copies.append(
            pltpu.async_copy(
                table_hbm.at[idx_v.at[pl.ds(j * CHUNK, CHUNK)]],
                rows_v.at[pl.ds(j * CHUNK, CHUNK)],
                sem,
            )
        )
    for c in copies:
        c.wait()

    pltpu.sync_copy(rows_v, out_hbm.at[pl.ds(base, B_PER_W)])


@jax.jit
def kernel(table, inputs):
    tokens = inputs.reshape(B).astype(jnp.int32)
    mesh = plsc.VectorSubcoreMesh(core_axis_name="c", subcore_axis_name="s")
    run = pl.kernel(
        _sc_body,
        out_type=jax.ShapeDtypeStruct((B, EMB), jnp.float32),
        mesh=mesh,
        scratch_types=[
            pltpu.VMEM((B_PER_W,), jnp.int32),
            pltpu.VMEM((B_PER_W, EMB), jnp.float32),
            pltpu.SemaphoreType.DMA,
        ],
    )
    return run(table, tokens)
